# baseline (speedup 1.0000x reference)
import functools

import jax
import jax.numpy as jnp
from jax import lax
from jax.experimental import pallas as pl
from jax.experimental.pallas import tpu as pltpu
from jax.experimental.pallas import tpu_sc as plsc

_N = 10000
_E = 320000
_D = 128
_H = 256
_NC = 2
_NS = 16
_K = 128
_EPAD = 323584
_NCHK = _EPAD // _K
_CPT = _NCHK // _NS
_CPW = _NCHK // (_NC * _NS)
_NDP = 10240
_RA = 632
_RL = _N - 15 * _RA

_GDN = lax.GatherDimensionNumbers(
    offset_dims=(), collapsed_slice_dims=(0,), start_index_map=(0,))

_mesh = plsc.VectorSubcoreMesh(
    core_axis_name="c", subcore_axis_name="s", num_cores=_NC, num_subcores=_NS)


def _splat(vec16, e):
  idx = jnp.full((16, 1), e, jnp.int32)
  return lax.gather(vec16, idx, _GDN, (1,),
                    mode=lax.GatherScatterMode.PROMISE_IN_BOUNDS)


def _per_tile_rows(t, fn):
  @pl.when(t < 15)
  def _():
    fn(_RA, t * _RA)

  @pl.when(t == 15)
  def _():
    fn(_RL, 15 * _RA)


def _zero_slab(zbuf, acc, n, r0):
  def body(i, _):
    pltpu.sync_copy(zbuf, acc.at[pl.ds(r0 + i * 8, 8)])
    return 0
  lax.fori_loop(0, n // 8, body, 0)


@functools.partial(
    pl.kernel,
    out_type=(
        jax.ShapeDtypeStruct((6 * _N, 128), jnp.float32),
        jax.ShapeDtypeStruct((6 * _NDP,), jnp.float32),
    ),
    mesh=_mesh,
    scratch_types=[
        pltpu.VMEM_SHARED((_N, 128), jnp.float32),
        pltpu.VMEM_SHARED((_NDP,), jnp.float32),
        pltpu.VMEM_SHARED((_NDP,), jnp.float32),
        pltpu.VMEM_SHARED((_NDP,), jnp.float32),
        pltpu.VMEM((_K,), jnp.int32),
        pltpu.VMEM((_K,), jnp.int32),
        pltpu.VMEM((_K,), jnp.float32),
        pltpu.VMEM((_K,), jnp.float32),
        pltpu.VMEM((_K,), jnp.float32),
        pltpu.VMEM((_K, 128), jnp.float32),
        pltpu.VMEM((_K, 128), jnp.float32),
        pltpu.VMEM((8, 128), jnp.float32),
        pltpu.VMEM((640,), jnp.float32),
        pltpu.SemaphoreType.DMA,
    ],
)
def _sc_layer1(xfull, src2d, dst2d, ew_h, em1_h, em2_h, agg1, degp,
               acc, d0, d1, d2,
               src_v, dst_v, ew_v, em_v, we_v, rows, msg, zbuf, dzbuf, sem):
  c = lax.axis_index("c")
  t = lax.axis_index("s")
  wid = c * _NS + t

  def zinit(i, _):
    zbuf[i, pl.ds(0, 16)] = jnp.zeros((16,), jnp.float32)
    for s in range(1, 8):
      zbuf[i, pl.ds(s * 16, 16)] = jnp.zeros((16,), jnp.float32)
    return 0
  lax.fori_loop(0, 8, zinit, 0)
  def zdeg(i, _):
    dzbuf[pl.ds(i * 16, 16)] = jnp.zeros((16,), jnp.float32)
    return 0
  lax.fori_loop(0, 40, zdeg, 0)
  for dd in (d0, d1, d2):
    pltpu.sync_copy(dzbuf, dd.at[pl.ds(t * 640, 640)])

  for v, dv in enumerate((d0, d1, d2)):
    _per_tile_rows(t, lambda n, r0: _zero_slab(zbuf, acc, n, r0))
    plsc.subcore_barrier()

    def sweep(j, _):
      ch = wid * _CPW + j
      pltpu.sync_copy(src2d.at[ch], src_v)
      pltpu.sync_copy(dst2d.at[ch], dst_v)
      pltpu.sync_copy(ew_h.at[pl.ds(ch * _K, _K)], ew_v)
      if v == 1:
        pltpu.sync_copy(em1_h.at[pl.ds(ch * _K, _K)], em_v)
      elif v == 2:
        pltpu.sync_copy(em2_h.at[pl.ds(ch * _K, _K)], em_v)
      def prod(g, _):
        w = ew_v[pl.ds(g * 16, 16)]
        if v > 0:
          w = w * em_v[pl.ds(g * 16, 16)]
        we_v[pl.ds(g * 16, 16)] = w
        return 0
      lax.fori_loop(0, _K // 16, prod, 0)
      pltpu.async_copy(xfull.at[src_v], rows, sem).wait()
      def grp(g, _):
        w16 = we_v[pl.ds(g * 16, 16)]
        for e in range(16):
          eg = g * 16 + e
          w = _splat(w16, e)
          for s in range(8):
            msg[eg, pl.ds(s * 16, 16)] = rows[eg, pl.ds(s * 16, 16)] * w
        return 0
      lax.fori_loop(0, _K // 16, grp, 0)
      pltpu.sync_copy(msg, acc.at[dst_v], add=True)
      pltpu.sync_copy(we_v, dv.at[dst_v], add=True)
      return 0
    lax.fori_loop(0, _CPW, sweep, 0)

    plsc.subcore_barrier()
    base = (v * _NC + c) * _N
    _per_tile_rows(t, lambda n, r0, base=base: pltpu.sync_copy(
        acc.at[pl.ds(r0, n)], agg1.at[pl.ds(base + r0, n)]))

  for v, dd in enumerate((d0, d1, d2)):
    off = (c * 3 + v) * _NDP + t * 640
    pltpu.sync_copy(dd.at[pl.ds(t * 640, 640)], degp.at[pl.ds(off, 640)])


@functools.partial(
    pl.kernel,
    out_type=jax.ShapeDtypeStruct((6 * _N, 128), jnp.float32),
    mesh=_mesh,
    scratch_types=[
        pltpu.VMEM_SHARED((_N, 128), jnp.float32),
        pltpu.VMEM((_K,), jnp.int32),
        pltpu.VMEM((_K,), jnp.int32),
        pltpu.VMEM((_K,), jnp.int32),
        pltpu.VMEM((_K,), jnp.float32),
        pltpu.VMEM((_K,), jnp.float32),
        pltpu.VMEM((_K,), jnp.float32),
        pltpu.VMEM((_K, 128), jnp.float32),
        pltpu.VMEM((_K, 128), jnp.float32),
        pltpu.VMEM((8, 128), jnp.float32),
        pltpu.SemaphoreType.DMA,
    ],
)
def _sc_layer2(hflat, src2d, dst2d, ew_h, em1_h, em2_h, agg2,
               acc, src_v, srcs_v, dst_v, ew_v, em_v, we_v,
               rows, msg, zbuf, sem):
  c = lax.axis_index("c")
  t = lax.axis_index("s")

  def zinit(i, _):
    for s in range(8):
      zbuf[i, pl.ds(s * 16, 16)] = jnp.zeros((16,), jnp.float32)
    return 0
  lax.fori_loop(0, 8, zinit, 0)

  def shift(offset):
    def body(g, _):
      srcs_v[pl.ds(g * 16, 16)] = src_v[pl.ds(g * 16, 16)] + offset
      return 0
    lax.fori_loop(0, _K // 16, body, 0)

  for v in range(3):
    _per_tile_rows(t, lambda n, r0: _zero_slab(zbuf, acc, n, r0))
    plsc.subcore_barrier()

    def sweep(j, _):
      ch = t * _CPT + j
      pltpu.sync_copy(src2d.at[ch], src_v)
      pltpu.sync_copy(dst2d.at[ch], dst_v)
      pltpu.sync_copy(ew_h.at[pl.ds(ch * _K, _K)], ew_v)
      if v == 1:
        pltpu.sync_copy(em1_h.at[pl.ds(ch * _K, _K)], em_v)
      elif v == 2:
        pltpu.sync_copy(em2_h.at[pl.ds(ch * _K, _K)], em_v)
      def prod(g, _):
        w = ew_v[pl.ds(g * 16, 16)]
        if v > 0:
          w = w * em_v[pl.ds(g * 16, 16)]
        we_v[pl.ds(g * 16, 16)] = w
        return 0
      lax.fori_loop(0, _K // 16, prod, 0)
      shift((v * _NC + c) * _N)
      pltpu.async_copy(hflat.at[srcs_v], rows, sem).wait()
      def grp(g, _):
        w16 = we_v[pl.ds(g * 16, 16)]
        for e in range(16):
          eg = g * 16 + e
          w = _splat(w16, e)
          for s in range(8):
            msg[eg, pl.ds(s * 16, 16)] = rows[eg, pl.ds(s * 16, 16)] * w
        return 0
      lax.fori_loop(0, _K // 16, grp, 0)
      pltpu.sync_copy(msg, acc.at[dst_v], add=True)
      return 0
    lax.fori_loop(0, _CPT, sweep, 0)

    plsc.subcore_barrier()
    base = (v * _NC + c) * _N
    _per_tile_rows(t, lambda n, r0, base=base: pltpu.sync_copy(
        acc.at[pl.ds(r0, n)], agg2.at[pl.ds(base + r0, n)]))


_BLK = 2000


def _deg_recip(deg_ref):
  v = pl.program_id(0)
  d = deg_ref[...]
  li = lax.broadcasted_iota(jnp.int32, d.shape, 1)
  dv = jnp.sum(jnp.where((li == v) | (li == v + 3), d, 0.0), axis=1)
  return 1.0 / jnp.clip(dv, 1e-6, None)


def _tc1_body(agg_ref, deg_ref, w_ref, b_ref, mask_ref, out_ref):
  m = agg_ref[0, 0] + agg_ref[0, 1]
  r = _deg_recip(deg_ref)
  mk = mask_ref[...]
  ri = lax.broadcasted_iota(jnp.int32, mk.shape, 0)
  mv = jnp.sum(jnp.where(ri == pl.program_id(0), mk, 0.0), axis=0)
  w = w_ref[...] * mv[:, None]
  h = jnp.dot(m, w, preferred_element_type=jnp.float32)
  h = jax.nn.relu(h * r[:, None] + b_ref[0][None, :])
  out_ref[0, 0] = h[:, :128]
  out_ref[0, 1] = h[:, 128:]


def _tc2_body(agg_ref, deg_ref, w_ref, b_ref, out_ref):
  m = jnp.concatenate([agg_ref[0, 0], agg_ref[0, 1]], axis=1)
  r = _deg_recip(deg_ref)
  z = jnp.dot(m, w_ref[...], preferred_element_type=jnp.float32)
  out_ref[0] = z * r[:, None] + b_ref[0][None, :]


def kernel(x, edge_index, edge_weight, feat_mask1, edge_mask1, feat_mask2,
           edge_mask2, W1, b1, W2, b2):
  f32 = jnp.float32
  src = edge_index[0]
  dst = edge_index[1]
  npad = _EPAD - _E
  spread = (jnp.arange(npad, dtype=jnp.int32) * 37) % _N
  src2d = jnp.concatenate([src, spread]).reshape(_NCHK, _K)
  dst2d = jnp.concatenate([dst, spread]).reshape(_NCHK, _K)
  zpad = jnp.zeros((npad,), f32)
  ew = jnp.concatenate([edge_weight, zpad])
  em1 = jnp.concatenate([edge_mask1.astype(f32), zpad])
  em2 = jnp.concatenate([edge_mask2.astype(f32), zpad])

  agg1, degp = _sc_layer1(x, src2d, dst2d, ew, em1, em2)
  agg1 = agg1.reshape(3, 2, _N, 128)
  deg = degp.reshape(6, _NDP)[:, :_N].T

  masks = jnp.stack([jnp.ones((_D,), f32),
                     feat_mask1.astype(f32), feat_mask2.astype(f32)])

  hh = pl.pallas_call(
      _tc1_body,
      grid=(3, _N // _BLK),
      in_specs=[
          pl.BlockSpec((1, 2, _BLK, 128), lambda v, i: (v, 0, i, 0)),
          pl.BlockSpec((_BLK, 6), lambda v, i: (i, 0)),
          pl.BlockSpec((_D, _H), lambda v, i: (0, 0)),
          pl.BlockSpec((1, _H), lambda v, i: (0, 0)),
          pl.BlockSpec((3, _D), lambda v, i: (0, 0)),
      ],
      out_specs=pl.BlockSpec((1, 2, _BLK, 128), lambda v, i: (v, 0, i, 0)),
      out_shape=jax.ShapeDtypeStruct((3, 2, _N, 128), f32),
  )(agg1, deg, W1, b1.reshape(1, _H), masks)

  agg2 = _sc_layer2(hh.reshape(6 * _N, 128), src2d, dst2d, ew, em1, em2)
  agg2 = agg2.reshape(3, 2, _N, 128)

  z3 = pl.pallas_call(
      _tc2_body,
      grid=(3, _N // _BLK),
      in_specs=[
          pl.BlockSpec((1, 2, _BLK, 128), lambda v, i: (v, 0, i, 0)),
          pl.BlockSpec((_BLK, 6), lambda v, i: (i, 0)),
          pl.BlockSpec((_H, _H), lambda v, i: (0, 0)),
          pl.BlockSpec((1, _H), lambda v, i: (0, 0)),
      ],
      out_specs=pl.BlockSpec((1, _BLK, _H), lambda v, i: (v, i, 0)),
      out_shape=jax.ShapeDtypeStruct((3, _N, _H), f32),
  )(agg2, deg, W2, b2.reshape(1, _H))

  return (z3[0], z3[1], z3[2])


# speedup vs baseline: 2.9239x; 2.9239x over previous
import functools

import jax
import jax.numpy as jnp
from jax import lax
from jax.experimental import pallas as pl
from jax.experimental.pallas import tpu as pltpu
from jax.experimental.pallas import tpu_sc as plsc

_N = 10000
_E = 320000
_D = 128
_H = 256
_NC = 2
_NS = 16
_K = 128
_EPAD = 327680
_NCHK = _EPAD // _K
_CPT = _NCHK // _NS
_CPW = _NCHK // (_NC * _NS)
_NDP = 10240
_RA = 632
_RL = _N - 15 * _RA

_GDN = lax.GatherDimensionNumbers(
    offset_dims=(), collapsed_slice_dims=(0,), start_index_map=(0,))

_mesh = plsc.VectorSubcoreMesh(
    core_axis_name="c", subcore_axis_name="s", num_cores=_NC, num_subcores=_NS)


def _splat(vec16, e):
  idx = jnp.full((16, 1), e, jnp.int32)
  return lax.gather(vec16, idx, _GDN, (1,),
                    mode=lax.GatherScatterMode.PROMISE_IN_BOUNDS)


def _per_tile_rows(t, fn):
  @pl.when(t < 15)
  def _():
    fn(_RA, t * _RA)

  @pl.when(t == 15)
  def _():
    fn(_RL, 15 * _RA)


def _zero_slab(zbuf, acc, n, r0):
  def body(i, _):
    pltpu.sync_copy(zbuf, acc.at[pl.ds(r0 + i * 8, 8)])
    return 0
  lax.fori_loop(0, n // 8, body, 0)


def _f32(x16):
  return lax.bitcast_convert_type(x16, jnp.float32)


def _pipelined_sweep(*, v, nc, chunk0, edmat, table, acc, dv,
                     edata, srcs, we, rows, semL, semG, semS, shift):
  def fire_L(j, sl):
    pltpu.async_copy(edmat.at[chunk0 + j], edata[sl], semL[sl])

  def wait_L(j, sl):
    pltpu.make_async_copy(edmat.at[chunk0 + j], edata[sl], semL[sl]).wait()

  def prod(j, sl, ssl):
    ed = edata[sl]
    def body(g, _):
      w = _f32(ed[2, pl.ds(g * 16, 16)])
      if v == 1:
        w = w * _f32(ed[3, pl.ds(g * 16, 16)])
      elif v == 2:
        w = w * _f32(ed[4, pl.ds(g * 16, 16)])
      we[sl][pl.ds(g * 16, 16)] = w
      if srcs is not None:
        srcs[ssl][pl.ds(g * 16, 16)] = ed[0, pl.ds(g * 16, 16)] + shift
      return 0
    lax.fori_loop(0, _K // 16, body, 0)

  def gidx(sl, ssl):
    return srcs[ssl] if srcs is not None else edata[sl].at[0]

  def fire_G(j, sl, ssl, rl):
    pltpu.async_copy(table.at[gidx(sl, ssl)], rows[rl], semG[rl])

  def wait_G(j, sl, ssl, rl):
    pltpu.make_async_copy(table.at[gidx(sl, ssl)], rows[rl], semG[rl]).wait()

  def compute(j, sl, rl):
    r = rows[rl]
    def body(g, _):
      w16 = we[sl][pl.ds(g * 16, 16)]
      for e in range(16):
        eg = g * 16 + e
        w = _splat(w16, e)
        for s in range(8):
          r[eg, pl.ds(s * 16, 16)] = r[eg, pl.ds(s * 16, 16)] * w
      return 0
    lax.fori_loop(0, _K // 16, body, 0)

  def fire_S(j, sl, rl):
    dref = edata[sl].at[1]
    pltpu.async_copy(rows[rl], acc.at[dref], semS[rl], add=True)
    if dv is not None:
      pltpu.async_copy(we[sl], dv.at[dref], semS[rl], add=True)

  def wait_S(j, sl, rl):
    dref = edata[sl].at[1]
    pltpu.make_async_copy(rows[rl], acc.at[dref], semS[rl]).wait()
    if dv is not None:
      pltpu.make_async_copy(we[sl], dv.at[dref], semS[rl]).wait()

  def step(j, tj):
    sl, sl1, sl2 = j % 4, (j + 1) % 4, (j + 2) % 4
    rl, rl1 = j % 2, (j + 1) % 2
    if j + 2 < nc:
      fire_L(tj + 2, sl2)
    if j >= 1:
      wait_S(j - 1, (j - 1) % 4, rl1)
    if j + 1 < nc:
      wait_L(tj + 1, sl1)
      prod(j + 1, sl1, rl1)
      fire_G(tj + 1, sl1, rl1, rl1)
    wait_G(tj, sl, rl, rl)
    compute(j, sl, rl)
    fire_S(tj, sl, rl)

  fire_L(0, 0)
  fire_L(1, 1)
  wait_L(0, 0)
  prod(0, 0, 0)
  fire_G(0, 0, 0, 0)
  step(0, jnp.int32(0))
  step(1, jnp.int32(1))

  def main(jj, _):
    base = 2 + jj * 4
    for u in range(4):
      step(2 + u, base + u)
    return 0
  lax.fori_loop(0, (nc - 4) // 4, main, 0)

  step(nc - 2, jnp.int32(nc - 2))
  step(nc - 1, jnp.int32(nc - 1))
  wait_S(nc - 1, (nc - 1) % 4, (nc - 1) % 2)


_EDGE_SCRATCH = [
    pltpu.VMEM((8, _K), jnp.int32),
    pltpu.VMEM((8, _K), jnp.int32),
    pltpu.VMEM((8, _K), jnp.int32),
    pltpu.VMEM((8, _K), jnp.int32),
    pltpu.VMEM((_K,), jnp.float32),
    pltpu.VMEM((_K,), jnp.float32),
    pltpu.VMEM((_K,), jnp.float32),
    pltpu.VMEM((_K,), jnp.float32),
    pltpu.VMEM((_K, 128), jnp.float32),
    pltpu.VMEM((_K, 128), jnp.float32),
    pltpu.VMEM((8, 128), jnp.float32),
    pltpu.SemaphoreType.DMA,
    pltpu.SemaphoreType.DMA,
    pltpu.SemaphoreType.DMA,
    pltpu.SemaphoreType.DMA,
    pltpu.SemaphoreType.DMA,
    pltpu.SemaphoreType.DMA,
    pltpu.SemaphoreType.DMA,
    pltpu.SemaphoreType.DMA,
]


@functools.partial(
    pl.kernel,
    out_type=(
        jax.ShapeDtypeStruct((6 * _N, 128), jnp.float32),
        jax.ShapeDtypeStruct((6 * _NDP,), jnp.float32),
    ),
    mesh=_mesh,
    scratch_types=[
        pltpu.VMEM_SHARED((_N, 128), jnp.float32),
        pltpu.VMEM_SHARED((_NDP,), jnp.float32),
        pltpu.VMEM_SHARED((_NDP,), jnp.float32),
        pltpu.VMEM_SHARED((_NDP,), jnp.float32),
        pltpu.VMEM((640,), jnp.float32),
    ] + _EDGE_SCRATCH,
)
def _sc_layer1(xfull, edmat, agg1, degp,
               acc, d0, d1, d2, dzbuf,
               ed0, ed1, ed2, ed3, we0, we1, we2, we3, rw0, rw1, zbuf,
               sL0, sL1, sL2, sL3, sG0, sG1, sS0, sS1):
  c = lax.axis_index("c")
  t = lax.axis_index("s")
  wid = c * _NS + t

  def zinit(i, _):
    for s in range(8):
      zbuf[i, pl.ds(s * 16, 16)] = jnp.zeros((16,), jnp.float32)
    return 0
  lax.fori_loop(0, 8, zinit, 0)
  def zdeg(i, _):
    dzbuf[pl.ds(i * 16, 16)] = jnp.zeros((16,), jnp.float32)
    return 0
  lax.fori_loop(0, 40, zdeg, 0)
  for dd in (d0, d1, d2):
    pltpu.sync_copy(dzbuf, dd.at[pl.ds(t * 640, 640)])

  for v, dvz in enumerate((d0, d1, d2)):
    _per_tile_rows(t, lambda n, r0: _zero_slab(zbuf, acc, n, r0))
    plsc.subcore_barrier()

    _pipelined_sweep(
        v=v, nc=_CPW, chunk0=wid * _CPW, edmat=edmat, table=xfull,
        acc=acc, dv=dvz,
        edata=(ed0, ed1, ed2, ed3), srcs=None, we=(we0, we1, we2, we3),
        rows=(rw0, rw1), semL=(sL0, sL1, sL2, sL3), semG=(sG0, sG1),
        semS=(sS0, sS1), shift=0)

    plsc.subcore_barrier()
    base = (v * _NC + c) * _N
    _per_tile_rows(t, lambda n, r0, base=base: pltpu.sync_copy(
        acc.at[pl.ds(r0, n)], agg1.at[pl.ds(base + r0, n)]))

  for v, dd in enumerate((d0, d1, d2)):
    off = (c * 3 + v) * _NDP + t * 640
    pltpu.sync_copy(dd.at[pl.ds(t * 640, 640)], degp.at[pl.ds(off, 640)])


@functools.partial(
    pl.kernel,
    out_type=jax.ShapeDtypeStruct((6 * _N, 128), jnp.float32),
    mesh=_mesh,
    scratch_types=[
        pltpu.VMEM_SHARED((_N, 128), jnp.float32),
        pltpu.VMEM((_K,), jnp.int32),
        pltpu.VMEM((_K,), jnp.int32),
    ] + _EDGE_SCRATCH,
)
def _sc_layer2(hflat, edmat, agg2,
               acc, ss0, ss1,
               ed0, ed1, ed2, ed3, we0, we1, we2, we3, rw0, rw1, zbuf,
               sL0, sL1, sL2, sL3, sG0, sG1, sS0, sS1):
  c = lax.axis_index("c")
  t = lax.axis_index("s")

  def zinit(i, _):
    for s in range(8):
      zbuf[i, pl.ds(s * 16, 16)] = jnp.zeros((16,), jnp.float32)
    return 0
  lax.fori_loop(0, 8, zinit, 0)

  for v in range(3):
    _per_tile_rows(t, lambda n, r0: _zero_slab(zbuf, acc, n, r0))
    plsc.subcore_barrier()

    _pipelined_sweep(
        v=v, nc=_CPT, chunk0=t * _CPT, edmat=edmat, table=hflat,
        acc=acc, dv=None,
        edata=(ed0, ed1, ed2, ed3), srcs=(ss0, ss1), we=(we0, we1, we2, we3),
        rows=(rw0, rw1), semL=(sL0, sL1, sL2, sL3), semG=(sG0, sG1),
        semS=(sS0, sS1), shift=(v * _NC + c) * _N)

    plsc.subcore_barrier()
    base = (v * _NC + c) * _N
    _per_tile_rows(t, lambda n, r0, base=base: pltpu.sync_copy(
        acc.at[pl.ds(r0, n)], agg2.at[pl.ds(base + r0, n)]))


_BLK = 2000


def _deg_recip(deg_ref):
  v = pl.program_id(0)
  d = deg_ref[...]
  li = lax.broadcasted_iota(jnp.int32, d.shape, 1)
  dv = jnp.sum(jnp.where((li == v) | (li == v + 3), d, 0.0), axis=1)
  return 1.0 / jnp.clip(dv, 1e-6, None)


def _tc1_body(agg_ref, deg_ref, w_ref, b_ref, mask_ref, out_ref):
  m = agg_ref[0, 0] + agg_ref[0, 1]
  r = _deg_recip(deg_ref)
  mk = mask_ref[...]
  ri = lax.broadcasted_iota(jnp.int32, mk.shape, 0)
  mv = jnp.sum(jnp.where(ri == pl.program_id(0), mk, 0.0), axis=0)
  w = w_ref[...] * mv[:, None]
  h = jnp.dot(m, w, preferred_element_type=jnp.float32)
  h = jax.nn.relu(h * r[:, None] + b_ref[0][None, :])
  out_ref[0, 0] = h[:, :128]
  out_ref[0, 1] = h[:, 128:]


def _tc2_body(agg_ref, deg_ref, w_ref, b_ref, out_ref):
  m = jnp.concatenate([agg_ref[0, 0], agg_ref[0, 1]], axis=1)
  r = _deg_recip(deg_ref)
  z = jnp.dot(m, w_ref[...], preferred_element_type=jnp.float32)
  out_ref[0] = z * r[:, None] + b_ref[0][None, :]


def kernel(x, edge_index, edge_weight, feat_mask1, edge_mask1, feat_mask2,
           edge_mask2, W1, b1, W2, b2):
  f32 = jnp.float32
  i32 = jnp.int32
  src = edge_index[0]
  dst = edge_index[1]
  npad = _EPAD - _E
  spread = (jnp.arange(npad, dtype=i32) * 37) % _N
  zpad = jnp.zeros((npad,), f32)

  def as_chunks(a):
    return a.reshape(_NCHK, 1, _K)

  edmat = jnp.concatenate([
      as_chunks(jnp.concatenate([src, spread])),
      as_chunks(jnp.concatenate([dst, spread])),
      as_chunks(lax.bitcast_convert_type(
          jnp.concatenate([edge_weight, zpad]), i32)),
      as_chunks(lax.bitcast_convert_type(
          jnp.concatenate([edge_mask1.astype(f32), zpad]), i32)),
      as_chunks(lax.bitcast_convert_type(
          jnp.concatenate([edge_mask2.astype(f32), zpad]), i32)),
      jnp.zeros((_NCHK, 3, _K), i32),
  ], axis=1)

  agg1, degp = _sc_layer1(x, edmat)
  agg1 = agg1.reshape(3, 2, _N, 128)
  deg = degp.reshape(6, _NDP)[:, :_N].T

  masks = jnp.stack([jnp.ones((_D,), f32),
                     feat_mask1.astype(f32), feat_mask2.astype(f32)])

  hh = pl.pallas_call(
      _tc1_body,
      grid=(3, _N // _BLK),
      in_specs=[
          pl.BlockSpec((1, 2, _BLK, 128), lambda v, i: (v, 0, i, 0)),
          pl.BlockSpec((_BLK, 6), lambda v, i: (i, 0)),
          pl.BlockSpec((_D, _H), lambda v, i: (0, 0)),
          pl.BlockSpec((1, _H), lambda v, i: (0, 0)),
          pl.BlockSpec((3, _D), lambda v, i: (0, 0)),
      ],
      out_specs=pl.BlockSpec((1, 2, _BLK, 128), lambda v, i: (v, 0, i, 0)),
      out_shape=jax.ShapeDtypeStruct((3, 2, _N, 128), f32),
  )(agg1, deg, W1, b1.reshape(1, _H), masks)

  agg2 = _sc_layer2(hh.reshape(6 * _N, 128), edmat)
  agg2 = agg2.reshape(3, 2, _N, 128)

  z3 = pl.pallas_call(
      _tc2_body,
      grid=(3, _N // _BLK),
      in_specs=[
          pl.BlockSpec((1, 2, _BLK, 128), lambda v, i: (v, 0, i, 0)),
          pl.BlockSpec((_BLK, 6), lambda v, i: (i, 0)),
          pl.BlockSpec((_H, _H), lambda v, i: (0, 0)),
          pl.BlockSpec((1, _H), lambda v, i: (0, 0)),
      ],
      out_specs=pl.BlockSpec((1, _BLK, _H), lambda v, i: (v, i, 0)),
      out_shape=jax.ShapeDtypeStruct((3, _N, _H), f32),
  )(agg2, deg, W2, b2.reshape(1, _H))

  return (z3[0], z3[1], z3[2])


# speedup vs baseline: 2.9787x; 1.0187x over previous
import functools

import jax
import jax.numpy as jnp
from jax import lax
from jax.experimental import pallas as pl
from jax.experimental.pallas import tpu as pltpu
from jax.experimental.pallas import tpu_sc as plsc

_N = 10000
_E = 320000
_D = 128
_H = 256
_NC = 2
_NS = 16
_K = 80
_EPAD = 327680
_NCHK = _EPAD // _K
_CPT = _NCHK // _NS
_CPW = _NCHK // (_NC * _NS)
_NDP = 10240
_RA = 632
_RL = _N - 15 * _RA

_GDN = lax.GatherDimensionNumbers(
    offset_dims=(), collapsed_slice_dims=(0,), start_index_map=(0,))

_mesh = plsc.VectorSubcoreMesh(
    core_axis_name="c", subcore_axis_name="s", num_cores=_NC, num_subcores=_NS)


def _splat(vec16, e):
  idx = jnp.full((16, 1), e, jnp.int32)
  return lax.gather(vec16, idx, _GDN, (1,),
                    mode=lax.GatherScatterMode.PROMISE_IN_BOUNDS)


def _per_tile_rows(t, fn):
  @pl.when(t < 15)
  def _():
    fn(_RA, t * _RA)

  @pl.when(t == 15)
  def _():
    fn(_RL, 15 * _RA)


def _zero_slab(zbuf, acc, n, r0):
  def body(i, _):
    pltpu.sync_copy(zbuf, acc.at[pl.ds(r0 + i * 8, 8)])
    return 0
  lax.fori_loop(0, n // 8, body, 0)


def _f32(x16):
  return lax.bitcast_convert_type(x16, jnp.float32)


def _pipelined_sweep(*, v, nc, chunk0, edmat, table, acc, dv,
                     edata, srcs, we, rows, semL, semG, semS, shift):
  def fire_L(j, sl):
    pltpu.async_copy(edmat.at[chunk0 + j], edata[sl], semL[sl])

  def wait_L(j, sl):
    pltpu.make_async_copy(edmat.at[chunk0 + j], edata[sl], semL[sl]).wait()

  def prod(j, sl, ssl):
    ed = edata[sl]
    def body(g, _):
      w = _f32(ed[2, pl.ds(g * 16, 16)])
      if v == 1:
        w = w * _f32(ed[3, pl.ds(g * 16, 16)])
      elif v == 2:
        w = w * _f32(ed[4, pl.ds(g * 16, 16)])
      we[sl][pl.ds(g * 16, 16)] = w
      if srcs is not None:
        srcs[ssl][pl.ds(g * 16, 16)] = ed[0, pl.ds(g * 16, 16)] + shift
      return 0
    lax.fori_loop(0, _K // 16, body, 0)

  def gidx(sl, ssl):
    return srcs[ssl] if srcs is not None else edata[sl].at[0]

  def fire_G(j, sl, ssl, rl):
    pltpu.async_copy(table.at[gidx(sl, ssl)], rows[rl], semG[rl])

  def wait_G(j, sl, ssl, rl):
    pltpu.make_async_copy(table.at[gidx(sl, ssl)], rows[rl], semG[rl]).wait()

  def compute(j, sl, rl):
    r = rows[rl]
    def body(g, _):
      w16 = we[sl][pl.ds(g * 16, 16)]
      for e in range(16):
        eg = g * 16 + e
        w = _splat(w16, e)
        for s in range(8):
          r[eg, pl.ds(s * 16, 16)] = r[eg, pl.ds(s * 16, 16)] * w
      return 0
    lax.fori_loop(0, _K // 16, body, 0)

  def fire_S(j, sl, rl):
    dref = edata[sl].at[1]
    pltpu.async_copy(rows[rl], acc.at[dref], semS[rl], add=True)
    if dv is not None:
      pltpu.async_copy(we[sl], dv.at[dref], semS[rl], add=True)

  def wait_S(j, sl, rl):
    dref = edata[sl].at[1]
    pltpu.make_async_copy(rows[rl], acc.at[dref], semS[rl]).wait()
    if dv is not None:
      pltpu.make_async_copy(we[sl], dv.at[dref], semS[rl]).wait()

  def step(j, tj):
    sl, sl1, sl2 = j % 4, (j + 1) % 4, (j + 2) % 4
    if j >= 2:
      wait_S(j - 2, (j - 2) % 4, (j - 2) % 4)
    if j + 2 < nc:
      fire_L(tj + 2, sl2)
    if j + 1 < nc:
      wait_L(tj + 1, sl1)
      prod(j + 1, sl1, sl1)
      fire_G(tj + 1, sl1, sl1, sl1)
    wait_G(tj, sl, sl, sl)
    compute(j, sl, sl)
    fire_S(tj, sl, sl)

  fire_L(0, 0)
  fire_L(1, 1)
  wait_L(0, 0)
  prod(0, 0, 0)
  fire_G(0, 0, 0, 0)
  step(0, jnp.int32(0))
  step(1, jnp.int32(1))

  def main(jj, _):
    base = 2 + jj * 4
    for u in range(4):
      step(2 + u, base + u)
    return 0
  lax.fori_loop(0, (nc - 4) // 4, main, 0)

  step(nc - 2, jnp.int32(nc - 2))
  step(nc - 1, jnp.int32(nc - 1))
  wait_S(nc - 2, (nc - 2) % 4, (nc - 2) % 4)
  wait_S(nc - 1, (nc - 1) % 4, (nc - 1) % 4)


_EDGE_SCRATCH = (
    [pltpu.VMEM((8, _K), jnp.int32)] * 4 +
    [pltpu.VMEM((_K,), jnp.float32)] * 4 +
    [pltpu.VMEM((_K, 128), jnp.float32)] * 4 +
    [pltpu.VMEM((8, 128), jnp.float32)] +
    [pltpu.SemaphoreType.DMA] * 12
)


@functools.partial(
    pl.kernel,
    out_type=(
        jax.ShapeDtypeStruct((6 * _N, 128), jnp.float32),
        jax.ShapeDtypeStruct((6 * _NDP,), jnp.float32),
    ),
    mesh=_mesh,
    scratch_types=[
        pltpu.VMEM_SHARED((_N, 128), jnp.float32),
        pltpu.VMEM_SHARED((_NDP,), jnp.float32),
        pltpu.VMEM_SHARED((_NDP,), jnp.float32),
        pltpu.VMEM_SHARED((_NDP,), jnp.float32),
        pltpu.VMEM((640,), jnp.float32),
    ] + _EDGE_SCRATCH,
)
def _sc_layer1(xfull, edmat, agg1, degp,
               acc, d0, d1, d2, dzbuf,
               ed0, ed1, ed2, ed3, we0, we1, we2, we3,
               rw0, rw1, rw2, rw3, zbuf,
               sL0, sL1, sL2, sL3, sG0, sG1, sG2, sG3, sS0, sS1, sS2, sS3):
  c = lax.axis_index("c")
  t = lax.axis_index("s")
  wid = c * _NS + t

  def zinit(i, _):
    for s in range(8):
      zbuf[i, pl.ds(s * 16, 16)] = jnp.zeros((16,), jnp.float32)
    return 0
  lax.fori_loop(0, 8, zinit, 0)
  def zdeg(i, _):
    dzbuf[pl.ds(i * 16, 16)] = jnp.zeros((16,), jnp.float32)
    return 0
  lax.fori_loop(0, 40, zdeg, 0)
  for dd in (d0, d1, d2):
    pltpu.sync_copy(dzbuf, dd.at[pl.ds(t * 640, 640)])

  for v, dvz in enumerate((d0, d1, d2)):
    _per_tile_rows(t, lambda n, r0: _zero_slab(zbuf, acc, n, r0))
    plsc.subcore_barrier()

    _pipelined_sweep(
        v=v, nc=_CPW, chunk0=wid * _CPW, edmat=edmat, table=xfull,
        acc=acc, dv=dvz,
        edata=(ed0, ed1, ed2, ed3), srcs=None, we=(we0, we1, we2, we3),
        rows=(rw0, rw1, rw2, rw3), semL=(sL0, sL1, sL2, sL3),
        semG=(sG0, sG1, sG2, sG3), semS=(sS0, sS1, sS2, sS3), shift=0)

    plsc.subcore_barrier()
    base = (v * _NC + c) * _N
    _per_tile_rows(t, lambda n, r0, base=base: pltpu.sync_copy(
        acc.at[pl.ds(r0, n)], agg1.at[pl.ds(base + r0, n)]))

  for v, dd in enumerate((d0, d1, d2)):
    off = (c * 3 + v) * _NDP + t * 640
    pltpu.sync_copy(dd.at[pl.ds(t * 640, 640)], degp.at[pl.ds(off, 640)])


@functools.partial(
    pl.kernel,
    out_type=jax.ShapeDtypeStruct((6 * _N, 128), jnp.float32),
    mesh=_mesh,
    scratch_types=[
        pltpu.VMEM_SHARED((_N, 128), jnp.float32),
        pltpu.VMEM((_K,), jnp.int32),
        pltpu.VMEM((_K,), jnp.int32),
        pltpu.VMEM((_K,), jnp.int32),
        pltpu.VMEM((_K,), jnp.int32),
    ] + _EDGE_SCRATCH,
)
def _sc_layer2(hflat, edmat, agg2,
               acc, ss0, ss1, ss2, ss3,
               ed0, ed1, ed2, ed3, we0, we1, we2, we3,
               rw0, rw1, rw2, rw3, zbuf,
               sL0, sL1, sL2, sL3, sG0, sG1, sG2, sG3, sS0, sS1, sS2, sS3):
  c = lax.axis_index("c")
  t = lax.axis_index("s")

  def zinit(i, _):
    for s in range(8):
      zbuf[i, pl.ds(s * 16, 16)] = jnp.zeros((16,), jnp.float32)
    return 0
  lax.fori_loop(0, 8, zinit, 0)

  for v in range(3):
    _per_tile_rows(t, lambda n, r0: _zero_slab(zbuf, acc, n, r0))
    plsc.subcore_barrier()

    _pipelined_sweep(
        v=v, nc=_CPT, chunk0=t * _CPT, edmat=edmat, table=hflat,
        acc=acc, dv=None,
        edata=(ed0, ed1, ed2, ed3), srcs=(ss0, ss1, ss2, ss3),
        we=(we0, we1, we2, we3), rows=(rw0, rw1, rw2, rw3),
        semL=(sL0, sL1, sL2, sL3), semG=(sG0, sG1, sG2, sG3),
        semS=(sS0, sS1, sS2, sS3), shift=(v * _NC + c) * _N)

    plsc.subcore_barrier()
    base = (v * _NC + c) * _N
    _per_tile_rows(t, lambda n, r0, base=base: pltpu.sync_copy(
        acc.at[pl.ds(r0, n)], agg2.at[pl.ds(base + r0, n)]))


_BLK = 2000


def _deg_recip(deg_ref):
  v = pl.program_id(0)
  d = deg_ref[...]
  li = lax.broadcasted_iota(jnp.int32, d.shape, 1)
  dv = jnp.sum(jnp.where((li == v) | (li == v + 3), d, 0.0), axis=1)
  return 1.0 / jnp.clip(dv, 1e-6, None)


def _tc1_body(agg_ref, deg_ref, w_ref, b_ref, mask_ref, out_ref):
  m = agg_ref[0, 0] + agg_ref[0, 1]
  r = _deg_recip(deg_ref)
  mk = mask_ref[...]
  ri = lax.broadcasted_iota(jnp.int32, mk.shape, 0)
  mv = jnp.sum(jnp.where(ri == pl.program_id(0), mk, 0.0), axis=0)
  w = w_ref[...] * mv[:, None]
  h = jnp.dot(m, w, preferred_element_type=jnp.float32)
  h = jax.nn.relu(h * r[:, None] + b_ref[0][None, :])
  out_ref[0, 0] = h[:, :128]
  out_ref[0, 1] = h[:, 128:]


def _tc2_body(agg_ref, deg_ref, w_ref, b_ref, out_ref):
  m = jnp.concatenate([agg_ref[0, 0], agg_ref[0, 1]], axis=1)
  r = _deg_recip(deg_ref)
  z = jnp.dot(m, w_ref[...], preferred_element_type=jnp.float32)
  out_ref[0] = z * r[:, None] + b_ref[0][None, :]


def kernel(x, edge_index, edge_weight, feat_mask1, edge_mask1, feat_mask2,
           edge_mask2, W1, b1, W2, b2):
  f32 = jnp.float32
  i32 = jnp.int32
  src = edge_index[0]
  dst = edge_index[1]
  npad = _EPAD - _E
  spread = (jnp.arange(npad, dtype=i32) * 37) % _N
  zpad = jnp.zeros((npad,), f32)

  def as_chunks(a):
    return a.reshape(_NCHK, 1, _K)

  edmat = jnp.concatenate([
      as_chunks(jnp.concatenate([src, spread])),
      as_chunks(jnp.concatenate([dst, spread])),
      as_chunks(lax.bitcast_convert_type(
          jnp.concatenate([edge_weight, zpad]), i32)),
      as_chunks(lax.bitcast_convert_type(
          jnp.concatenate([edge_mask1.astype(f32), zpad]), i32)),
      as_chunks(lax.bitcast_convert_type(
          jnp.concatenate([edge_mask2.astype(f32), zpad]), i32)),
      jnp.zeros((_NCHK, 3, _K), i32),
  ], axis=1)

  agg1, degp = _sc_layer1(x, edmat)
  agg1 = agg1.reshape(3, 2, _N, 128)
  deg = degp.reshape(6, _NDP)[:, :_N].T

  masks = jnp.stack([jnp.ones((_D,), f32),
                     feat_mask1.astype(f32), feat_mask2.astype(f32)])

  hh = pl.pallas_call(
      _tc1_body,
      grid=(3, _N // _BLK),
      in_specs=[
          pl.BlockSpec((1, 2, _BLK, 128), lambda v, i: (v, 0, i, 0)),
          pl.BlockSpec((_BLK, 6), lambda v, i: (i, 0)),
          pl.BlockSpec((_D, _H), lambda v, i: (0, 0)),
          pl.BlockSpec((1, _H), lambda v, i: (0, 0)),
          pl.BlockSpec((3, _D), lambda v, i: (0, 0)),
      ],
      out_specs=pl.BlockSpec((1, 2, _BLK, 128), lambda v, i: (v, 0, i, 0)),
      out_shape=jax.ShapeDtypeStruct((3, 2, _N, 128), f32),
  )(agg1, deg, W1, b1.reshape(1, _H), masks)

  agg2 = _sc_layer2(hh.reshape(6 * _N, 128), edmat)
  agg2 = agg2.reshape(3, 2, _N, 128)

  z3 = pl.pallas_call(
      _tc2_body,
      grid=(3, _N // _BLK),
      in_specs=[
          pl.BlockSpec((1, 2, _BLK, 128), lambda v, i: (v, 0, i, 0)),
          pl.BlockSpec((_BLK, 6), lambda v, i: (i, 0)),
          pl.BlockSpec((_H, _H), lambda v, i: (0, 0)),
          pl.BlockSpec((1, _H), lambda v, i: (0, 0)),
      ],
      out_specs=pl.BlockSpec((1, _BLK, _H), lambda v, i: (v, i, 0)),
      out_shape=jax.ShapeDtypeStruct((3, _N, _H), f32),
  )(agg2, deg, W2, b2.reshape(1, _H))

  return (z3[0], z3[1], z3[2])
